# baseline (speedup 1.0000x reference)
import functools

import jax
import jax.numpy as jnp
from jax import lax
from jax.experimental import pallas as pl
from jax.experimental.pallas import tpu as pltpu
from jax.experimental.pallas import tpu_sc as plsc

NC = 2
NS = 16
NW = NC * NS
L = 16
CHUNK = 128
BM = 256


def _attn_terms(xp, a_src_ref, a_dst_ref):
    s = jnp.sum(xp * a_src_ref[...], axis=1)
    d = jnp.sum(xp * a_dst_ref[...], axis=1)
    e = s + d
    e = jnp.where(e < 0, 0.2 * e, e)
    w0 = jnp.exp(e)
    return s, d, w0


def _prep_body(x_ref, w_ref, a_src_ref, a_dst_ref,
               xp_ref, als_ref, ald_ref, w0_ref, num0_ref):
    xp = jnp.dot(x_ref[...], w_ref[...], preferred_element_type=jnp.float32)
    s, d, w0 = _attn_terms(xp, a_src_ref, a_dst_ref)
    xp_ref[...] = xp
    als_ref[...] = s
    ald_ref[...] = d
    w0_ref[...] = w0
    num0_ref[...] = xp * w0[:, None]


def _combine(acc_ref, num0_ref, w0_ref, D):
    acc = acc_ref[0] + acc_ref[1]
    num = acc[:, :D] + num0_ref[...]
    den = acc[:, D] + w0_ref[...]
    return num / (den[:, None] + 1e-16)


def _mid_body(acc_ref, num0_ref, w0_ref, b_ref, w2_ref, a_src_ref, a_dst_ref,
              xp_ref, als_ref, ald_ref, w0o_ref, num0o_ref):
    D = num0_ref.shape[-1]
    h = _combine(acc_ref, num0_ref, w0_ref, D) + b_ref[...]
    h = jnp.where(h > 0, h, jnp.exp(h) - 1.0)
    xp = jnp.dot(h, w2_ref[...], preferred_element_type=jnp.float32)
    s, d, w0 = _attn_terms(xp, a_src_ref, a_dst_ref)
    xp_ref[...] = xp
    als_ref[...] = s
    ald_ref[...] = d
    w0o_ref[...] = w0
    num0o_ref[...] = xp * w0[:, None]


def _final_body(acc_ref, num0_ref, w0_ref, b_ref, out_ref):
    D = num0_ref.shape[-1]
    o = _combine(acc_ref, num0_ref, w0_ref, D) + b_ref[...]
    m = jnp.max(o, axis=1, keepdims=True)
    z = o - m
    out_ref[...] = z - jnp.log(jnp.sum(jnp.exp(z), axis=1, keepdims=True))


def _prep_call(Npad, Fin, D, x, W, a_src, a_dst):
    grid = (Npad // BM,)
    row2 = pl.BlockSpec((BM, D), lambda i: (i, 0))
    row1 = pl.BlockSpec((BM,), lambda i: (i,))
    return pl.pallas_call(
        _prep_body,
        grid=grid,
        in_specs=[
            pl.BlockSpec((BM, Fin), lambda i: (i, 0)),
            pl.BlockSpec((Fin, D), lambda i: (0, 0)),
            pl.BlockSpec((1, D), lambda i: (0, 0)),
            pl.BlockSpec((1, D), lambda i: (0, 0)),
        ],
        out_specs=[row2, row1, row1, row1, row2],
        out_shape=[
            jax.ShapeDtypeStruct((Npad, D), jnp.float32),
            jax.ShapeDtypeStruct((Npad,), jnp.float32),
            jax.ShapeDtypeStruct((Npad,), jnp.float32),
            jax.ShapeDtypeStruct((Npad,), jnp.float32),
            jax.ShapeDtypeStruct((Npad, D), jnp.float32),
        ],
    )(x, W, a_src, a_dst)


def _mid_call(Npad, D, acc, num0, w0, b, W2, a_src, a_dst):
    grid = (Npad // BM,)
    row2 = pl.BlockSpec((BM, D), lambda i: (i, 0))
    row1 = pl.BlockSpec((BM,), lambda i: (i,))
    return pl.pallas_call(
        _mid_body,
        grid=grid,
        in_specs=[
            pl.BlockSpec((NC, BM, D + L), lambda i: (0, i, 0)),
            row2,
            row1,
            pl.BlockSpec((1, D), lambda i: (0, 0)),
            pl.BlockSpec((D, D), lambda i: (0, 0)),
            pl.BlockSpec((1, D), lambda i: (0, 0)),
            pl.BlockSpec((1, D), lambda i: (0, 0)),
        ],
        out_specs=[row2, row1, row1, row1, row2],
        out_shape=[
            jax.ShapeDtypeStruct((Npad, D), jnp.float32),
            jax.ShapeDtypeStruct((Npad,), jnp.float32),
            jax.ShapeDtypeStruct((Npad,), jnp.float32),
            jax.ShapeDtypeStruct((Npad,), jnp.float32),
            jax.ShapeDtypeStruct((Npad, D), jnp.float32),
        ],
    )(acc, num0, w0, b, W2, a_src, a_dst)


def _final_call(Npad, D, acc, num0, w0, b):
    grid = (Npad // BM,)
    return pl.pallas_call(
        _final_body,
        grid=grid,
        in_specs=[
            pl.BlockSpec((NC, BM, D + L), lambda i: (0, i, 0)),
            pl.BlockSpec((BM, D), lambda i: (i, 0)),
            pl.BlockSpec((BM,), lambda i: (i,)),
            pl.BlockSpec((1, D), lambda i: (0, 0)),
        ],
        out_specs=pl.BlockSpec((BM, D), lambda i: (i, 0)),
        out_shape=jax.ShapeDtypeStruct((Npad, D), jnp.float32),
    )(acc, num0, w0, b)


@functools.lru_cache(maxsize=None)
def _sc_warmup_kernel():
    mesh = plsc.VectorSubcoreMesh(core_axis_name="c", subcore_axis_name="s")

    @functools.partial(
        pl.kernel,
        out_type=jax.ShapeDtypeStruct((L,), jnp.int32),
        mesh=mesh,
        compiler_params=pltpu.CompilerParams(needs_layout_passes=False,
                                             use_tc_tiling_on_sc=False),
        scratch_types=[pltpu.VMEM((L,), jnp.int32)],
    )
    def k(out_hbm, buf):
        cid = lax.axis_index("c")
        sid = lax.axis_index("s")

        @pl.when(jnp.logical_and(cid == 0, sid == 0))
        def _():
            buf[...] = jnp.zeros((L,), jnp.int32)
            pltpu.sync_copy(buf, out_hbm)

    return k


@functools.lru_cache(maxsize=None)
def _sc_edge_kernel(Npad, D, EW, nchunks):
    DW = D + L
    RT = Npad // NW
    mesh = plsc.VectorSubcoreMesh(core_axis_name="c", subcore_axis_name="s")

    RB = 2

    @functools.partial(
        pl.kernel,
        out_type=jax.ShapeDtypeStruct((NC, Npad, DW), jnp.float32),
        mesh=mesh,
        compiler_params=pltpu.CompilerParams(needs_layout_passes=False,
                                             use_tc_tiling_on_sc=False),
        scratch_types=[
            pltpu.VMEM((Npad,), jnp.float32),
            pltpu.VMEM((Npad,), jnp.float32),
            pltpu.VMEM((nchunks, CHUNK), jnp.int32),
            pltpu.VMEM((nchunks, CHUNK), jnp.int32),
            pltpu.VMEM((RB, CHUNK, D), jnp.float32),
            pltpu.VMEM((RB, CHUNK, DW), jnp.float32),
            pltpu.VMEM_SHARED((Npad, DW), jnp.float32),
            pltpu.SemaphoreType.DMA,
            pltpu.SemaphoreType.DMA,
            pltpu.SemaphoreType.DMA,
            pltpu.SemaphoreType.DMA,
        ],
    )
    def k(src_hbm, dst_hbm, als_hbm, ald_hbm, xp_hbm, acc_hbm,
          asrc_t, adst_t, idx_s, idx_d, rows_g, rows_s, acc_sh,
          sg0, sg1, ss0, ss1):
        cid = lax.axis_index("c")
        sid = lax.axis_index("s")
        wid = sid * NC + cid
        sg = (sg0, sg1)
        ss = (ss0, ss1)

        zero = jnp.zeros((L,), jnp.float32)
        for i in range(CHUNK):
            for jj in range(DW // L):
                rows_s[0, i, pl.ds(jj * L, L)] = zero
        r0 = sid * RT
        for off in range(0, RT, CHUNK):
            n = min(CHUNK, RT - off)
            pltpu.sync_copy(rows_s.at[0, pl.ds(0, n)],
                            acc_sh.at[pl.ds(r0 + off, n)])

        pltpu.sync_copy(als_hbm, asrc_t)
        pltpu.sync_copy(ald_hbm, adst_t)
        plsc.subcore_barrier()

        pltpu.sync_copy(src_hbm.at[wid], idx_s)
        pltpu.sync_copy(dst_hbm.at[wid], idx_d)

        def start_gather(rb, g):
            pltpu.async_copy(xp_hbm.at[idx_s.at[g]], rows_g.at[rb], sg[rb])

        def wait_gather(rb, g):
            pltpu.make_async_copy(xp_hbm.at[idx_s.at[g]], rows_g.at[rb],
                                  sg[rb]).wait()

        def start_scatter(rb, g):
            pltpu.async_copy(rows_s.at[rb], acc_sh.at[idx_d.at[g]], ss[rb],
                             add=True)

        def wait_scatter(rb, g):
            pltpu.make_async_copy(rows_s.at[rb], acc_sh.at[idx_d.at[g]],
                                  ss[rb]).wait()

        for g in range(RB - 1):
            start_gather(g, g)

        lanes = jnp.arange(L, dtype=jnp.int32)

        def quad(q, _):
            for b in range(RB):
                g = RB * q + b
                gv = jnp.full((L,), g, jnp.int32)
                w16s = []
                for j in range(CHUNK // L):
                    cols = lanes + (j * L)
                    sv = plsc.load_gather(idx_s, [gv, cols])
                    dv = plsc.load_gather(idx_d, [gv, cols])
                    e = (plsc.load_gather(asrc_t, [sv])
                         + plsc.load_gather(adst_t, [dv]))
                    e = jnp.where(e < 0, 0.2 * e, e)
                    w16s.append(jnp.exp(e))

                nb = (b + RB - 1) % RB
                if b == 0:
                    start_gather(nb, g + RB - 1)
                else:
                    @pl.when(q < nquads - 1)
                    def _():
                        start_gather(nb, g + RB - 1)

                @pl.when(q > 0)
                def _():
                    wait_scatter(b, jnp.maximum(g - RB, 0))

                wait_gather(b, g)
                for j in range(CHUNK // L):
                    w16 = w16s[j]
                    for t in range(L):
                        i = j * L + t
                        wv = jnp.full((L,), w16[t], jnp.float32)
                        for jj in range(D // L):
                            rows_s[b, i, pl.ds(jj * L, L)] = (
                                rows_g[b, i, pl.ds(jj * L, L)] * wv)
                        rows_s[b, i, pl.ds(D, L)] = wv
                start_scatter(b, g)
            return 0

        nquads = nchunks // RB
        lax.fori_loop(0, nquads, quad, 0)
        for b in range(RB):
            wait_scatter(b, nchunks - RB + b)
        plsc.subcore_barrier()
        pltpu.sync_copy(acc_sh.at[pl.ds(r0, RT)],
                        acc_hbm.at[cid, pl.ds(r0, RT)])

    return k


def kernel(x, edge_index, W1, a_src1, a_dst1, b1, W2, a_src2, a_dst2, b2):
    N, Fin = x.shape
    H, Hd = a_src1.shape
    O = a_src2.shape[1]
    D = H * Hd
    E = edge_index.shape[1]

    Npad = -(-(N + 1) // BM) * BM
    nchunks = 4 * -(-E // (NW * CHUNK * 4))
    EW = nchunks * CHUNK
    Epad = EW * NW

    xpad = jnp.zeros((Npad, Fin), jnp.float32).at[:N].set(x)
    warm = _sc_warmup_kernel()()
    srcp = (jnp.zeros((Epad,), jnp.int32).at[:E].set(edge_index[0] + warm[0])
            .reshape(NW, nchunks, CHUNK))
    dstp = (jnp.full((Epad,), N, jnp.int32).at[:E].set(edge_index[1])
            .reshape(NW, nchunks, CHUNK))

    a_src1 = a_src1.reshape(1, D)
    a_dst1 = a_dst1.reshape(1, D)
    a_src2 = a_src2.reshape(1, O)
    a_dst2 = a_dst2.reshape(1, O)

    xp1, als1, ald1, w01, num01 = _prep_call(Npad, Fin, D, xpad, W1,
                                             a_src1, a_dst1)
    sc_edge = _sc_edge_kernel(Npad, D, EW, nchunks)
    acc1 = sc_edge(srcp, dstp, als1, ald1, xp1)
    xp2, als2, ald2, w02, num02 = _mid_call(Npad, D, acc1, num01, w01,
                                            b1.reshape(1, D), W2,
                                            a_src2, a_dst2)
    acc2 = sc_edge(srcp, dstp, als2, ald2, xp2)
    out = _final_call(Npad, O, acc2, num02, w02, b2.reshape(1, O))
    return out[:N]


# speedup vs baseline: 1.1072x; 1.1072x over previous
import functools

import jax
import jax.numpy as jnp
from jax import lax
from jax.experimental import pallas as pl
from jax.experimental.pallas import tpu as pltpu
from jax.experimental.pallas import tpu_sc as plsc

NC = 2
NS = 16
NW = NC * NS
L = 16
CHUNK = 128
BM = 256


def _attn_terms(xp, a_src_ref, a_dst_ref):
    s = jnp.sum(xp * a_src_ref[...], axis=1)
    d = jnp.sum(xp * a_dst_ref[...], axis=1)
    e = s + d
    e = jnp.where(e < 0, 0.2 * e, e)
    w0 = jnp.exp(e)
    return s, d, w0


def _prep_body(x_ref, w_ref, a_src_ref, a_dst_ref,
               xp_ref, als_ref, ald_ref, w0_ref, num0_ref):
    xp = jnp.dot(x_ref[...], w_ref[...], preferred_element_type=jnp.float32)
    s, d, w0 = _attn_terms(xp, a_src_ref, a_dst_ref)
    xp_ref[...] = xp
    als_ref[...] = s
    ald_ref[...] = d
    w0_ref[...] = w0
    num0_ref[...] = xp * w0[:, None]


def _combine(acc_ref, num0_ref, w0_ref, D):
    acc = acc_ref[0] + acc_ref[1]
    num = acc[:, :D] + num0_ref[...]
    den = acc[:, D] + w0_ref[...]
    return num / (den[:, None] + 1e-16)


def _mid_body(acc_ref, num0_ref, w0_ref, b_ref, w2_ref, a_src_ref, a_dst_ref,
              xp_ref, als_ref, ald_ref, w0o_ref, num0o_ref):
    D = num0_ref.shape[-1]
    h = _combine(acc_ref, num0_ref, w0_ref, D) + b_ref[...]
    h = jnp.where(h > 0, h, jnp.exp(h) - 1.0)
    xp = jnp.dot(h, w2_ref[...], preferred_element_type=jnp.float32)
    s, d, w0 = _attn_terms(xp, a_src_ref, a_dst_ref)
    xp_ref[...] = xp
    als_ref[...] = s
    ald_ref[...] = d
    w0o_ref[...] = w0
    num0o_ref[...] = xp * w0[:, None]


def _final_body(acc_ref, num0_ref, w0_ref, b_ref, out_ref):
    D = num0_ref.shape[-1]
    o = _combine(acc_ref, num0_ref, w0_ref, D) + b_ref[...]
    m = jnp.max(o, axis=1, keepdims=True)
    z = o - m
    out_ref[...] = z - jnp.log(jnp.sum(jnp.exp(z), axis=1, keepdims=True))


def _prep_call(Npad, Fin, D, x, W, a_src, a_dst):
    grid = (Npad // BM,)
    row2 = pl.BlockSpec((BM, D), lambda i: (i, 0))
    row1 = pl.BlockSpec((BM,), lambda i: (i,))
    return pl.pallas_call(
        _prep_body,
        grid=grid,
        in_specs=[
            pl.BlockSpec((BM, Fin), lambda i: (i, 0)),
            pl.BlockSpec((Fin, D), lambda i: (0, 0)),
            pl.BlockSpec((1, D), lambda i: (0, 0)),
            pl.BlockSpec((1, D), lambda i: (0, 0)),
        ],
        out_specs=[row2, row1, row1, row1, row2],
        out_shape=[
            jax.ShapeDtypeStruct((Npad, D), jnp.float32),
            jax.ShapeDtypeStruct((Npad,), jnp.float32),
            jax.ShapeDtypeStruct((Npad,), jnp.float32),
            jax.ShapeDtypeStruct((Npad,), jnp.float32),
            jax.ShapeDtypeStruct((Npad, D), jnp.float32),
        ],
    )(x, W, a_src, a_dst)


def _mid_call(Npad, D, acc, num0, w0, b, W2, a_src, a_dst):
    grid = (Npad // BM,)
    row2 = pl.BlockSpec((BM, D), lambda i: (i, 0))
    row1 = pl.BlockSpec((BM,), lambda i: (i,))
    return pl.pallas_call(
        _mid_body,
        grid=grid,
        in_specs=[
            pl.BlockSpec((NC, BM, D + L), lambda i: (0, i, 0)),
            row2,
            row1,
            pl.BlockSpec((1, D), lambda i: (0, 0)),
            pl.BlockSpec((D, D), lambda i: (0, 0)),
            pl.BlockSpec((1, D), lambda i: (0, 0)),
            pl.BlockSpec((1, D), lambda i: (0, 0)),
        ],
        out_specs=[row2, row1, row1, row1, row2],
        out_shape=[
            jax.ShapeDtypeStruct((Npad, D), jnp.float32),
            jax.ShapeDtypeStruct((Npad,), jnp.float32),
            jax.ShapeDtypeStruct((Npad,), jnp.float32),
            jax.ShapeDtypeStruct((Npad,), jnp.float32),
            jax.ShapeDtypeStruct((Npad, D), jnp.float32),
        ],
    )(acc, num0, w0, b, W2, a_src, a_dst)


def _final_call(Npad, D, acc, num0, w0, b):
    grid = (Npad // BM,)
    return pl.pallas_call(
        _final_body,
        grid=grid,
        in_specs=[
            pl.BlockSpec((NC, BM, D + L), lambda i: (0, i, 0)),
            pl.BlockSpec((BM, D), lambda i: (i, 0)),
            pl.BlockSpec((BM,), lambda i: (i,)),
            pl.BlockSpec((1, D), lambda i: (0, 0)),
        ],
        out_specs=pl.BlockSpec((BM, D), lambda i: (i, 0)),
        out_shape=jax.ShapeDtypeStruct((Npad, D), jnp.float32),
    )(acc, num0, w0, b)


@functools.lru_cache(maxsize=None)
def _sc_edge_kernel(Npad, D, EW, nchunks):
    DW = D + L
    RT = Npad // NW
    mesh = plsc.VectorSubcoreMesh(core_axis_name="c", subcore_axis_name="s")

    RB = 2

    @functools.partial(
        pl.kernel,
        out_type=jax.ShapeDtypeStruct((NC, Npad, DW), jnp.float32),
        mesh=mesh,
        compiler_params=pltpu.CompilerParams(needs_layout_passes=False,
                                             use_tc_tiling_on_sc=False),
        scratch_types=[
            pltpu.VMEM((Npad,), jnp.float32),
            pltpu.VMEM((Npad,), jnp.float32),
            pltpu.VMEM((nchunks, CHUNK), jnp.int32),
            pltpu.VMEM((nchunks, CHUNK), jnp.int32),
            pltpu.VMEM((RB, CHUNK, D), jnp.float32),
            pltpu.VMEM((RB, CHUNK, DW), jnp.float32),
            pltpu.VMEM_SHARED((Npad, DW), jnp.float32),
            pltpu.SemaphoreType.DMA,
            pltpu.SemaphoreType.DMA,
            pltpu.SemaphoreType.DMA,
            pltpu.SemaphoreType.DMA,
        ],
    )
    def k(src_hbm, dst_hbm, als_hbm, ald_hbm, xp_hbm, acc_hbm,
          asrc_t, adst_t, idx_s, idx_d, rows_g, rows_s, acc_sh,
          sg0, sg1, ss0, ss1):
        cid = lax.axis_index("c")
        sid = lax.axis_index("s")
        wid = sid * NC + cid
        sg = (sg0, sg1)
        ss = (ss0, ss1)

        zero = jnp.zeros((L,), jnp.float32)
        for i in range(CHUNK):
            for jj in range(DW // L):
                rows_s[0, i, pl.ds(jj * L, L)] = zero
        r0 = sid * RT
        for off in range(0, RT, CHUNK):
            n = min(CHUNK, RT - off)
            pltpu.sync_copy(rows_s.at[0, pl.ds(0, n)],
                            acc_sh.at[pl.ds(r0 + off, n)])

        pltpu.sync_copy(als_hbm, asrc_t)
        pltpu.sync_copy(ald_hbm, adst_t)
        plsc.subcore_barrier()

        pltpu.sync_copy(src_hbm.at[wid], idx_s)
        pltpu.sync_copy(dst_hbm.at[wid], idx_d)

        def start_gather(rb, g):
            pltpu.async_copy(xp_hbm.at[idx_s.at[g]], rows_g.at[rb], sg[rb])

        def wait_gather(rb, g):
            pltpu.make_async_copy(xp_hbm.at[idx_s.at[g]], rows_g.at[rb],
                                  sg[rb]).wait()

        def start_scatter(rb, g):
            pltpu.async_copy(rows_s.at[rb], acc_sh.at[idx_d.at[g]], ss[rb],
                             add=True)

        def wait_scatter(rb, g):
            pltpu.make_async_copy(rows_s.at[rb], acc_sh.at[idx_d.at[g]],
                                  ss[rb]).wait()

        for g in range(RB - 1):
            start_gather(g, g)

        lanes = jnp.arange(L, dtype=jnp.int32)

        def quad(q, _):
            for b in range(RB):
                g = RB * q + b
                gv = jnp.full((L,), g, jnp.int32)
                w16s = []
                for j in range(CHUNK // L):
                    cols = lanes + (j * L)
                    sv = plsc.load_gather(idx_s, [gv, cols])
                    dv = plsc.load_gather(idx_d, [gv, cols])
                    e = (plsc.load_gather(asrc_t, [sv])
                         + plsc.load_gather(adst_t, [dv]))
                    e = jnp.where(e < 0, 0.2 * e, e)
                    w16s.append(jnp.exp(e))

                nb = (b + RB - 1) % RB
                if b == 0:
                    start_gather(nb, g + RB - 1)
                else:
                    @pl.when(q < nquads - 1)
                    def _():
                        start_gather(nb, g + RB - 1)

                @pl.when(q > 0)
                def _():
                    wait_scatter(b, jnp.maximum(g - RB, 0))

                wait_gather(b, g)
                for j in range(CHUNK // L):
                    w16 = w16s[j]
                    for t in range(L):
                        i = j * L + t
                        wv = jnp.full((L,), w16[t], jnp.float32)
                        for jj in range(D // L):
                            rows_s[b, i, pl.ds(jj * L, L)] = (
                                rows_g[b, i, pl.ds(jj * L, L)] * wv)
                        rows_s[b, i, pl.ds(D, L)] = wv
                start_scatter(b, g)
            return 0

        nquads = nchunks // RB
        lax.fori_loop(0, nquads, quad, 0)
        for b in range(RB):
            wait_scatter(b, nchunks - RB + b)
        plsc.subcore_barrier()
        pltpu.sync_copy(acc_sh.at[pl.ds(r0, RT)],
                        acc_hbm.at[cid, pl.ds(r0, RT)])

    return k


def kernel(x, edge_index, W1, a_src1, a_dst1, b1, W2, a_src2, a_dst2, b2):
    N, Fin = x.shape
    H, Hd = a_src1.shape
    O = a_src2.shape[1]
    D = H * Hd
    E = edge_index.shape[1]

    Npad = -(-(N + 1) // BM) * BM
    nchunks = 4 * -(-E // (NW * CHUNK * 4))
    EW = nchunks * CHUNK
    Epad = EW * NW

    xpad = jnp.zeros((Npad, Fin), jnp.float32).at[:N].set(x)
    srcp = (jnp.zeros((Epad,), jnp.int32).at[:E].set(edge_index[0])
            .reshape(NW, nchunks, CHUNK))
    dstp = (jnp.full((Epad,), N, jnp.int32).at[:E].set(edge_index[1])
            .reshape(NW, nchunks, CHUNK))

    a_src1 = a_src1.reshape(1, D)
    a_dst1 = a_dst1.reshape(1, D)
    a_src2 = a_src2.reshape(1, O)
    a_dst2 = a_dst2.reshape(1, O)

    xp1, als1, ald1, w01, num01 = _prep_call(Npad, Fin, D, xpad, W1,
                                             a_src1, a_dst1)
    sc_edge = _sc_edge_kernel(Npad, D, EW, nchunks)
    acc1 = sc_edge(srcp, dstp, als1, ald1, xp1)
    xp2, als2, ald2, w02, num02 = _mid_call(Npad, D, acc1, num01, w01,
                                            b1.reshape(1, D), W2,
                                            a_src2, a_dst2)
    acc2 = sc_edge(srcp, dstp, als2, ald2, xp2)
    out = _final_call(Npad, O, acc2, num02, w02, b2.reshape(1, O))
    return out[:N]
